# baseline (speedup 1.0000x reference)
import functools

import jax
import jax.numpy as jnp
from jax import lax
from jax.experimental import pallas as pl
from jax.experimental.pallas import tpu as pltpu
from jax.experimental.pallas import tpu_sc as plsc

BANK_N = 1024
V_LEN = 16
HID = 4096
BS = 128
NSLOT = BS * V_LEN
FLAT_N = BANK_N * V_LEN

ROWS_PER_TILE = 16
_SC_S = 4
_TC_S = V_LEN - _SC_S


def _rotl(x, d):
    return lax.shift_left(x, jnp.uint32(d)) | lax.shift_right_logical(
        x, jnp.uint32(32 - d))


def _threefry_xor(x1):
    ks = (jnp.uint32(0), jnp.uint32(1), jnp.uint32(0x1BD11BDB))
    rot = ((13, 15, 26, 6), (17, 29, 16, 24))
    v0 = jnp.zeros_like(x1)
    v1 = x1 + ks[1]
    for i in range(5):
        for r in rot[i % 2]:
            v0 = v0 + v1
            v1 = _rotl(v1, r)
            v1 = v0 ^ v1
        v0 = v0 + ks[(i + 1) % 3]
        v1 = v1 + ks[(i + 2) % 3] + jnp.uint32(i + 1)
    return v0 ^ v1


_CHUNK = 512
_S_PER_TILE = 3
_NCHUNKS = FLAT_N // _CHUNK
_CBITS = 5
_CMASK = (1 << _CBITS) - 1


def _sample_body(vid_ref, out_ref):
    sq = pl.program_id(0)
    ib = pl.program_id(1)
    vid = vid_ref[0, :, 0].reshape(ROWS_PER_TILE, 1)
    row = lax.broadcasted_iota(jnp.uint32, (ROWS_PER_TILE, _CHUNK), 0)
    colc = lax.broadcasted_iota(jnp.uint32, (ROWS_PER_TILE, _CHUNK), 1)
    coli0 = colc.astype(jnp.int32)

    def chunk(c, keys):
        cu = c.astype(jnp.uint32)
        coli = coli0 + c * _CHUNK
        new = []
        for k in range(_S_PER_TILE):
            base = ((_SC_S + sq * _S_PER_TILE + k) * BS
                    + ib * ROWS_PER_TILE) * FLAT_N
            x1 = (jnp.uint32(base) + row * jnp.uint32(FLAT_N) + colc
                  + cu * jnp.uint32(_CHUNK))
            bits = _threefry_xor(x1)
            kk = (lax.shift_right_logical(bits, jnp.uint32(9 - _CBITS))
                  & jnp.uint32(0xFFFFFFFF ^ _CMASK)).astype(jnp.int32)
            kk = kk | (_NCHUNKS - 1 - c)
            banned = lax.shift_right_logical(coli, 4) == vid
            new.append(jnp.maximum(keys[k], jnp.where(banned, -1, kk)))
        return tuple(new)

    keyinit = jnp.full((ROWS_PER_TILE, _CHUNK), -1, jnp.int32)
    keys = lax.fori_loop(0, _NCHUNKS, chunk, (keyinit,) * _S_PER_TILE)
    for k in range(_S_PER_TILE):
        key = keys[k]
        mkey = jnp.max(key, axis=1, keepdims=True)
        j = (_NCHUNKS - 1 - (key & _CMASK)) * _CHUNK + coli0
        cand = jnp.where(key == mkey, j, FLAT_N)
        out_ref[k, :, 0] = jnp.min(cand, axis=1)


def _sample_call(vid_idx):
    grid = (_TC_S // _S_PER_TILE, BS // ROWS_PER_TILE)
    return pl.pallas_call(
        _sample_body,
        grid=grid,
        in_specs=[pl.BlockSpec((1, ROWS_PER_TILE, 1),
                               lambda sq, ib: (0, ib, 0))],
        out_specs=pl.BlockSpec((_S_PER_TILE, ROWS_PER_TILE, 1),
                               lambda sq, ib: (sq, ib, 0)),
        out_shape=jax.ShapeDtypeStruct((_TC_S, BS, 1), jnp.int32),
    )(vid_idx.reshape(1, BS, 1))


_SC_WORKERS = 32
_SC_SLOTS = _SC_S * BS
_SLOT_PER_TEC = _SC_SLOTS // _SC_WORKERS
_NSTREAM = 8
_SC_STEPS = FLAT_N // _NSTREAM // 16


_SCOLS = FLAT_N // _NSTREAM


def _sc_sample_body(vid_hbm, out_hbm, vid_v, res_v, sem):
    wid = lax.axis_index("s") * 2 + lax.axis_index("c")
    slot0 = wid * _SLOT_PER_TEC
    pltpu.sync_copy(vid_hbm.at[pl.ds(slot0 % BS, _SLOT_PER_TEC)], vid_v)
    vidv = vid_v[...]
    lane = lax.iota(jnp.int32, 16)
    slotbase = ((jnp.full((16,), slot0, jnp.int32) + lane)
                * FLAT_N).astype(jnp.uint32)

    def step(c, carry):
        vals, cids = carry
        newv, newc = [], []
        for ts in range(_NSTREAM):
            col = ts * _SCOLS + c
            x1 = slotbase + col.astype(jnp.uint32)
            bits = _threefry_xor(x1)
            u = lax.shift_right_logical(bits, jnp.uint32(9)).astype(jnp.int32)
            banned = vidv == lax.shift_right_logical(col, 4)
            u = jnp.where(banned, -1, u)
            better = u > vals[ts]
            newv.append(jnp.where(better, u, vals[ts]))
            newc.append(jnp.where(better, c, cids[ts]))
        return tuple(newv), tuple(newc)

    neg1 = jnp.full((16,), -1, jnp.int32)
    zero = jnp.zeros((16,), jnp.int32)
    vals, cids = lax.fori_loop(
        0, _SCOLS, step, ((neg1,) * _NSTREAM, (zero,) * _NSTREAM))
    bval, bj = vals[0], cids[0]
    for ts in range(1, _NSTREAM):
        better = vals[ts] > bval
        bval = jnp.where(better, vals[ts], bval)
        bj = jnp.where(better, cids[ts] + ts * _SCOLS, bj)
    res_v[...] = bj
    pltpu.sync_copy(res_v, out_hbm.at[pl.ds(slot0, _SLOT_PER_TEC)])


@functools.lru_cache(maxsize=1)
def _sc_sample():
    return functools.partial(
        pl.kernel,
        mesh=plsc.VectorSubcoreMesh(core_axis_name="c", subcore_axis_name="s"),
        out_type=jax.ShapeDtypeStruct((_SC_SLOTS,), jnp.int32),
        scratch_types=[
            pltpu.VMEM((16,), jnp.int32),
            pltpu.VMEM((16,), jnp.int32),
            pltpu.SemaphoreType.DMA,
        ],
    )(_sc_sample_body)


_ROWS_PER_W = NSLOT // _SC_WORKERS
_GROUP = 4


def _gather_blend_body(bank_hbm, idx_hbm, vid_hbm, bg_hbm, out_hbm,
                       idx_v, bg_v, bank_v, vid_v,
                       gsem0, gsem1, vsem0, vsem1, osem0, osem1):
    gsem = (gsem0, gsem1)
    vsem = (vsem0, vsem1)
    osem = (osem0, osem1)
    wid = lax.axis_index("s") * 2 + lax.axis_index("c")
    base = wid * _ROWS_PER_W
    ngroups = _ROWS_PER_W // _GROUP
    pltpu.sync_copy(idx_hbm.at[pl.ds(wid * ngroups, ngroups)], idx_v)
    pltpu.sync_copy(bg_hbm.at[pl.ds(base, _ROWS_PER_W)], bg_v)

    def issue(g, par):
        gc = pltpu.async_copy(
            bank_hbm.at[idx_v.at[g]], bank_v.at[par], gsem[par])
        vc = pltpu.async_copy(
            vid_hbm.at[pl.ds(base + g * _GROUP, _GROUP)], vid_v.at[par],
            vsem[par])
        return gc, vc

    pend = {0: issue(0, 0)}
    out_pend = {}
    for g in range(ngroups):
        par = g & 1
        if g + 1 < ngroups:
            if g - 1 >= 0:
                out_pend.pop(g - 1).wait()
            pend[g + 1] = issue(g + 1, 1 - par)
        gc, vc = pend.pop(g)
        gc.wait()
        vc.wait()
        for r in range(_GROUP):
            mvec = bg_v[g * _GROUP + r, :]
            inv = 1.0 - mvec

            def body(c, _):
                sl = pl.ds(c * 16, 16)
                bank_v[par, r, sl] = (vid_v[par, r, sl] * inv
                                      + bank_v[par, r, sl] * mvec)
                return 0

            lax.fori_loop(0, HID // 16, body, 0, unroll=8)
        out_pend[g] = pltpu.async_copy(
            bank_v.at[par], out_hbm.at[pl.ds(base + g * _GROUP, _GROUP)],
            osem[par])
    out_pend.pop(ngroups - 2).wait()
    out_pend.pop(ngroups - 1).wait()


@functools.lru_cache(maxsize=1)
def _gather_blend():
    return functools.partial(
        pl.kernel,
        mesh=plsc.VectorSubcoreMesh(core_axis_name="c", subcore_axis_name="s"),
        out_type=jax.ShapeDtypeStruct((NSLOT, HID), jnp.float32),
        scratch_types=[
            pltpu.VMEM((_ROWS_PER_W // _GROUP, _GROUP), jnp.int32),
            pltpu.VMEM((_ROWS_PER_W, 16), jnp.float32),
            pltpu.VMEM((2, _GROUP, HID), jnp.float32),
            pltpu.VMEM((2, _GROUP, HID), jnp.float32),
            pltpu.SemaphoreType.DMA,
            pltpu.SemaphoreType.DMA,
            pltpu.SemaphoreType.DMA,
            pltpu.SemaphoreType.DMA,
            pltpu.SemaphoreType.DMA,
            pltpu.SemaphoreType.DMA,
        ],
    )(_gather_blend_body)


def kernel(bg_mask, vid_feats, vid_idx, mem_bank):
    bs, v_len, hid = vid_feats.shape
    n_bank = mem_bank.shape[0]
    vid32 = vid_idx.astype(jnp.int32)
    sc_si = _sc_sample()(vid32)
    tc_si = _sample_call(vid32)
    lin = jnp.concatenate([sc_si, tc_si.reshape(_TC_S * bs)])
    sample_flat = lin.reshape(v_len, bs).T.reshape(bs * v_len)
    bank_flat = mem_bank.reshape(n_bank * v_len, hid)
    vid_flat = vid_feats.reshape(bs * v_len, hid)
    bg_wide = jnp.broadcast_to(bg_mask.reshape(bs * v_len, 1), (bs * v_len, 16))
    sample_2d = sample_flat.reshape(bs * v_len // _GROUP, _GROUP)
    out = _gather_blend()(bank_flat, sample_2d, vid_flat, bg_wide)
    return out.reshape(bs, v_len, hid)


# speedup vs baseline: 1.0356x; 1.0356x over previous
import functools

import jax
import jax.numpy as jnp
from jax import lax
from jax.experimental import pallas as pl
from jax.experimental.pallas import tpu as pltpu
from jax.experimental.pallas import tpu_sc as plsc

BANK_N = 1024
V_LEN = 16
HID = 4096
BS = 128
NSLOT = BS * V_LEN
FLAT_N = BANK_N * V_LEN

ROWS_PER_TILE = 16
_SC_S = 4
_TC_S = V_LEN - _SC_S


def _rotl(x, d):
    return lax.shift_left(x, jnp.uint32(d)) | lax.shift_right_logical(
        x, jnp.uint32(32 - d))


def _threefry_xor(x1):
    ks = (jnp.uint32(0), jnp.uint32(1), jnp.uint32(0x1BD11BDB))
    rot = ((13, 15, 26, 6), (17, 29, 16, 24))
    v0 = jnp.zeros_like(x1)
    v1 = x1 + ks[1]
    for i in range(5):
        for r in rot[i % 2]:
            v0 = v0 + v1
            v1 = _rotl(v1, r)
            v1 = v0 ^ v1
        v0 = v0 + ks[(i + 1) % 3]
        v1 = v1 + ks[(i + 2) % 3] + jnp.uint32(i + 1)
    return v0 ^ v1


_CHUNK = 512
_S_PER_TILE = 4
_NCHUNKS = FLAT_N // _CHUNK
_CBITS = 5
_CMASK = (1 << _CBITS) - 1


def _sample_body(vid_ref, out_ref):
    sq = pl.program_id(0)
    ib = pl.program_id(1)
    vid = vid_ref[0, :, 0].reshape(ROWS_PER_TILE, 1)
    row = lax.broadcasted_iota(jnp.uint32, (ROWS_PER_TILE, _CHUNK), 0)
    colc = lax.broadcasted_iota(jnp.uint32, (ROWS_PER_TILE, _CHUNK), 1)
    coli0 = colc.astype(jnp.int32)

    def chunk(c, keys):
        cu = c.astype(jnp.uint32)
        coli = coli0 + c * _CHUNK
        new = []
        for k in range(_S_PER_TILE):
            base = ((_SC_S + sq * _S_PER_TILE + k) * BS
                    + ib * ROWS_PER_TILE) * FLAT_N
            x1 = (jnp.uint32(base) + row * jnp.uint32(FLAT_N) + colc
                  + cu * jnp.uint32(_CHUNK))
            bits = _threefry_xor(x1)
            kk = (lax.shift_right_logical(bits, jnp.uint32(9 - _CBITS))
                  & jnp.uint32(0xFFFFFFFF ^ _CMASK)).astype(jnp.int32)
            kk = kk | (_NCHUNKS - 1 - c)
            banned = lax.shift_right_logical(coli, 4) == vid
            new.append(jnp.maximum(keys[k], jnp.where(banned, -1, kk)))
        return tuple(new)

    keyinit = jnp.full((ROWS_PER_TILE, _CHUNK), -1, jnp.int32)
    keys = lax.fori_loop(0, _NCHUNKS, chunk, (keyinit,) * _S_PER_TILE)
    for k in range(_S_PER_TILE):
        key = keys[k]
        mkey = jnp.max(key, axis=1, keepdims=True)
        j = (_NCHUNKS - 1 - (key & _CMASK)) * _CHUNK + coli0
        cand = jnp.where(key == mkey, j, FLAT_N)
        out_ref[k, :, 0] = jnp.min(cand, axis=1)


def _sample_call(vid_idx):
    grid = (_TC_S // _S_PER_TILE, BS // ROWS_PER_TILE)
    return pl.pallas_call(
        _sample_body,
        grid=grid,
        in_specs=[pl.BlockSpec((1, ROWS_PER_TILE, 1),
                               lambda sq, ib: (0, ib, 0))],
        out_specs=pl.BlockSpec((_S_PER_TILE, ROWS_PER_TILE, 1),
                               lambda sq, ib: (sq, ib, 0)),
        out_shape=jax.ShapeDtypeStruct((_TC_S, BS, 1), jnp.int32),
    )(vid_idx.reshape(1, BS, 1))


_SC_WORKERS = 32
_SC_SLOTS = _SC_S * BS
_SLOT_PER_TEC = _SC_SLOTS // _SC_WORKERS
_NSTREAM = 8
_SC_STEPS = FLAT_N // _NSTREAM // 16


_SCOLS = FLAT_N // _NSTREAM


def _sc_sample_body(vid_hbm, out_hbm, vid_v, res_v, sem):
    wid = lax.axis_index("s") * 2 + lax.axis_index("c")
    slot0 = wid * _SLOT_PER_TEC
    pltpu.sync_copy(vid_hbm.at[pl.ds(slot0 % BS, _SLOT_PER_TEC)], vid_v)
    vidv = vid_v[...]
    lane = lax.iota(jnp.int32, 16)
    slotbase = ((jnp.full((16,), slot0, jnp.int32) + lane)
                * FLAT_N).astype(jnp.uint32)

    def step(c, carry):
        vals, cids = carry
        newv, newc = [], []
        for ts in range(_NSTREAM):
            col = ts * _SCOLS + c
            x1 = slotbase + col.astype(jnp.uint32)
            bits = _threefry_xor(x1)
            u = lax.shift_right_logical(bits, jnp.uint32(9)).astype(jnp.int32)
            banned = vidv == lax.shift_right_logical(col, 4)
            u = jnp.where(banned, -1, u)
            better = u > vals[ts]
            newv.append(jnp.where(better, u, vals[ts]))
            newc.append(jnp.where(better, c, cids[ts]))
        return tuple(newv), tuple(newc)

    neg1 = jnp.full((16,), -1, jnp.int32)
    zero = jnp.zeros((16,), jnp.int32)
    vals, cids = lax.fori_loop(
        0, _SCOLS, step, ((neg1,) * _NSTREAM, (zero,) * _NSTREAM))
    bval, bj = vals[0], cids[0]
    for ts in range(1, _NSTREAM):
        better = vals[ts] > bval
        bval = jnp.where(better, vals[ts], bval)
        bj = jnp.where(better, cids[ts] + ts * _SCOLS, bj)
    res_v[...] = bj
    pltpu.sync_copy(res_v, out_hbm.at[pl.ds(slot0, _SLOT_PER_TEC)])


@functools.lru_cache(maxsize=1)
def _sc_sample():
    return functools.partial(
        pl.kernel,
        mesh=plsc.VectorSubcoreMesh(core_axis_name="c", subcore_axis_name="s"),
        out_type=jax.ShapeDtypeStruct((_SC_SLOTS,), jnp.int32),
        scratch_types=[
            pltpu.VMEM((16,), jnp.int32),
            pltpu.VMEM((16,), jnp.int32),
            pltpu.SemaphoreType.DMA,
        ],
    )(_sc_sample_body)


_ROWS_PER_W = NSLOT // _SC_WORKERS
_GROUP = 4


_NBUF = 3


def _gather_blend_body(bank_hbm, idx_hbm, vid_hbm, bg_hbm, out_hbm,
                       idx_v, bg_v, bank_v, vid_v,
                       gsem0, gsem1, gsem2, vsem0, vsem1, vsem2,
                       osem0, osem1, osem2):
    gsem = (gsem0, gsem1, gsem2)
    vsem = (vsem0, vsem1, vsem2)
    osem = (osem0, osem1, osem2)
    wid = lax.axis_index("s") * 2 + lax.axis_index("c")
    base = wid * _ROWS_PER_W
    ngroups = _ROWS_PER_W // _GROUP
    pltpu.sync_copy(idx_hbm.at[pl.ds(wid * ngroups, ngroups)], idx_v)
    pltpu.sync_copy(bg_hbm.at[pl.ds(base, _ROWS_PER_W)], bg_v)

    def issue(g):
        b = g % _NBUF
        gc = pltpu.async_copy(
            bank_hbm.at[idx_v.at[g]], bank_v.at[b], gsem[b])
        vc = pltpu.async_copy(
            vid_hbm.at[pl.ds(base + g * _GROUP, _GROUP)], vid_v.at[b],
            vsem[b])
        return gc, vc

    pend = {0: issue(0), 1: issue(1)}
    out_pend = {}
    for g in range(ngroups):
        b = g % _NBUF
        if g + 2 < ngroups:
            if g - 1 >= 0:
                out_pend.pop(g - 1).wait()
            pend[g + 2] = issue(g + 2)
        gc, vc = pend.pop(g)
        gc.wait()
        vc.wait()
        for r in range(_GROUP):
            mvec = bg_v[g * _GROUP + r, :]
            inv = 1.0 - mvec

            def body(c, _):
                sl = pl.ds(c * 16, 16)
                bank_v[b, r, sl] = (vid_v[b, r, sl] * inv
                                    + bank_v[b, r, sl] * mvec)
                return 0

            lax.fori_loop(0, HID // 16, body, 0, unroll=8)
        out_pend[g] = pltpu.async_copy(
            bank_v.at[b], out_hbm.at[pl.ds(base + g * _GROUP, _GROUP)],
            osem[b])
    for g in sorted(out_pend):
        out_pend[g].wait()


@functools.lru_cache(maxsize=1)
def _gather_blend():
    return functools.partial(
        pl.kernel,
        mesh=plsc.VectorSubcoreMesh(core_axis_name="c", subcore_axis_name="s"),
        out_type=jax.ShapeDtypeStruct((NSLOT, HID), jnp.float32),
        scratch_types=[
            pltpu.VMEM((_ROWS_PER_W // _GROUP, _GROUP), jnp.int32),
            pltpu.VMEM((_ROWS_PER_W, 16), jnp.float32),
            pltpu.VMEM((_NBUF, _GROUP, HID), jnp.float32),
            pltpu.VMEM((_NBUF, _GROUP, HID), jnp.float32),
        ] + [pltpu.SemaphoreType.DMA] * 9,
    )(_gather_blend_body)


def kernel(bg_mask, vid_feats, vid_idx, mem_bank):
    bs, v_len, hid = vid_feats.shape
    n_bank = mem_bank.shape[0]
    vid32 = vid_idx.astype(jnp.int32)
    sc_si = _sc_sample()(vid32)
    tc_si = _sample_call(vid32)
    lin = jnp.concatenate([sc_si, tc_si.reshape(_TC_S * bs)])
    sample_flat = lin.reshape(v_len, bs).T.reshape(bs * v_len)
    bank_flat = mem_bank.reshape(n_bank * v_len, hid)
    vid_flat = vid_feats.reshape(bs * v_len, hid)
    bg_wide = jnp.broadcast_to(bg_mask.reshape(bs * v_len, 1), (bs * v_len, 16))
    sample_2d = sample_flat.reshape(bs * v_len // _GROUP, _GROUP)
    out = _gather_blend()(bank_flat, sample_2d, vid_flat, bg_wide)
    return out.reshape(bs, v_len, hid)


# speedup vs baseline: 1.0427x; 1.0069x over previous
import functools

import jax
import jax.numpy as jnp
from jax import lax
from jax.experimental import pallas as pl
from jax.experimental.pallas import tpu as pltpu
from jax.experimental.pallas import tpu_sc as plsc

BANK_N = 1024
V_LEN = 16
HID = 4096
BS = 128
NSLOT = BS * V_LEN
FLAT_N = BANK_N * V_LEN

ROWS_PER_TILE = 32
_SC_S = 4
_TC_S = V_LEN - _SC_S


def _rotl(x, d):
    return lax.shift_left(x, jnp.uint32(d)) | lax.shift_right_logical(
        x, jnp.uint32(32 - d))


def _threefry_xor(x1):
    ks = (jnp.uint32(0), jnp.uint32(1), jnp.uint32(0x1BD11BDB))
    rot = ((13, 15, 26, 6), (17, 29, 16, 24))
    v0 = jnp.zeros_like(x1)
    v1 = x1 + ks[1]
    for i in range(5):
        for r in rot[i % 2]:
            v0 = v0 + v1
            v1 = _rotl(v1, r)
            v1 = v0 ^ v1
        v0 = v0 + ks[(i + 1) % 3]
        v1 = v1 + ks[(i + 2) % 3] + jnp.uint32(i + 1)
    return v0 ^ v1


_CHUNK = 256
_S_PER_TILE = 4
_NCHUNKS = FLAT_N // _CHUNK
_CBITS = 6
_CMASK = (1 << _CBITS) - 1


def _sample_body(vid_ref, out_ref):
    sq = pl.program_id(0)
    ib = pl.program_id(1)
    vid = vid_ref[0, :, 0].reshape(ROWS_PER_TILE, 1)
    row = lax.broadcasted_iota(jnp.uint32, (ROWS_PER_TILE, _CHUNK), 0)
    colc = lax.broadcasted_iota(jnp.uint32, (ROWS_PER_TILE, _CHUNK), 1)
    coli0 = colc.astype(jnp.int32)

    def chunk(c, keys):
        cu = c.astype(jnp.uint32)
        coli = coli0 + c * _CHUNK
        new = []
        for k in range(_S_PER_TILE):
            base = ((_SC_S + sq * _S_PER_TILE + k) * BS
                    + ib * ROWS_PER_TILE) * FLAT_N
            x1 = (jnp.uint32(base) + row * jnp.uint32(FLAT_N) + colc
                  + cu * jnp.uint32(_CHUNK))
            bits = _threefry_xor(x1)
            kk = (lax.shift_right_logical(bits, jnp.uint32(9 - _CBITS))
                  & jnp.uint32(0xFFFFFFFF ^ _CMASK)).astype(jnp.int32)
            kk = kk | (_NCHUNKS - 1 - c)
            banned = lax.shift_right_logical(coli, 4) == vid
            new.append(jnp.maximum(keys[k], jnp.where(banned, -1, kk)))
        return tuple(new)

    keyinit = jnp.full((ROWS_PER_TILE, _CHUNK), -1, jnp.int32)
    keys = lax.fori_loop(0, _NCHUNKS, chunk, (keyinit,) * _S_PER_TILE)
    for k in range(_S_PER_TILE):
        key = keys[k]
        mkey = jnp.max(key, axis=1, keepdims=True)
        j = (_NCHUNKS - 1 - (key & _CMASK)) * _CHUNK + coli0
        cand = jnp.where(key == mkey, j, FLAT_N)
        out_ref[k, :, 0] = jnp.min(cand, axis=1)


def _sample_call(vid_idx):
    grid = (_TC_S // _S_PER_TILE, BS // ROWS_PER_TILE)
    return pl.pallas_call(
        _sample_body,
        grid=grid,
        in_specs=[pl.BlockSpec((1, ROWS_PER_TILE, 1),
                               lambda sq, ib: (0, ib, 0))],
        out_specs=pl.BlockSpec((_S_PER_TILE, ROWS_PER_TILE, 1),
                               lambda sq, ib: (sq, ib, 0)),
        out_shape=jax.ShapeDtypeStruct((_TC_S, BS, 1), jnp.int32),
    )(vid_idx.reshape(1, BS, 1))


_SC_WORKERS = 32
_SC_SLOTS = _SC_S * BS
_SLOT_PER_TEC = _SC_SLOTS // _SC_WORKERS
_NSTREAM = 8
_SC_STEPS = FLAT_N // _NSTREAM // 16


_SCOLS = FLAT_N // _NSTREAM


def _sc_sample_body(vid_hbm, out_hbm, vid_v, res_v, sem):
    wid = lax.axis_index("s") * 2 + lax.axis_index("c")
    slot0 = wid * _SLOT_PER_TEC
    pltpu.sync_copy(vid_hbm.at[pl.ds(slot0 % BS, _SLOT_PER_TEC)], vid_v)
    vidv = vid_v[...]
    lane = lax.iota(jnp.int32, 16)
    slotbase = ((jnp.full((16,), slot0, jnp.int32) + lane)
                * FLAT_N).astype(jnp.uint32)

    def step(c, carry):
        vals, cids = carry
        newv, newc = [], []
        for ts in range(_NSTREAM):
            col = ts * _SCOLS + c
            x1 = slotbase + col.astype(jnp.uint32)
            bits = _threefry_xor(x1)
            u = lax.shift_right_logical(bits, jnp.uint32(9)).astype(jnp.int32)
            banned = vidv == lax.shift_right_logical(col, 4)
            u = jnp.where(banned, -1, u)
            better = u > vals[ts]
            newv.append(jnp.where(better, u, vals[ts]))
            newc.append(jnp.where(better, c, cids[ts]))
        return tuple(newv), tuple(newc)

    neg1 = jnp.full((16,), -1, jnp.int32)
    zero = jnp.zeros((16,), jnp.int32)
    vals, cids = lax.fori_loop(
        0, _SCOLS, step, ((neg1,) * _NSTREAM, (zero,) * _NSTREAM))
    bval, bj = vals[0], cids[0]
    for ts in range(1, _NSTREAM):
        better = vals[ts] > bval
        bval = jnp.where(better, vals[ts], bval)
        bj = jnp.where(better, cids[ts] + ts * _SCOLS, bj)
    res_v[...] = bj
    pltpu.sync_copy(res_v, out_hbm.at[pl.ds(slot0, _SLOT_PER_TEC)])


@functools.lru_cache(maxsize=1)
def _sc_sample():
    return functools.partial(
        pl.kernel,
        mesh=plsc.VectorSubcoreMesh(core_axis_name="c", subcore_axis_name="s"),
        out_type=jax.ShapeDtypeStruct((_SC_SLOTS,), jnp.int32),
        scratch_types=[
            pltpu.VMEM((16,), jnp.int32),
            pltpu.VMEM((16,), jnp.int32),
            pltpu.SemaphoreType.DMA,
        ],
    )(_sc_sample_body)


_ROWS_PER_W = NSLOT // _SC_WORKERS
_GROUP = 4


_NBUF = 3


def _gather_blend_body(bank_hbm, idx_hbm, vid_hbm, bg_hbm, out_hbm,
                       idx_v, bg_v, bank_v, vid_v,
                       gsem0, gsem1, gsem2, vsem0, vsem1, vsem2,
                       osem0, osem1, osem2):
    gsem = (gsem0, gsem1, gsem2)
    vsem = (vsem0, vsem1, vsem2)
    osem = (osem0, osem1, osem2)
    wid = lax.axis_index("s") * 2 + lax.axis_index("c")
    base = wid * _ROWS_PER_W
    ngroups = _ROWS_PER_W // _GROUP
    pltpu.sync_copy(idx_hbm.at[pl.ds(wid * ngroups, ngroups)], idx_v)
    pltpu.sync_copy(bg_hbm.at[pl.ds(base, _ROWS_PER_W)], bg_v)

    def issue(g):
        b = g % _NBUF
        gc = pltpu.async_copy(
            bank_hbm.at[idx_v.at[g]], bank_v.at[b], gsem[b])
        vc = pltpu.async_copy(
            vid_hbm.at[pl.ds(base + g * _GROUP, _GROUP)], vid_v.at[b],
            vsem[b])
        return gc, vc

    pend = {0: issue(0), 1: issue(1)}
    out_pend = {}
    for g in range(ngroups):
        b = g % _NBUF
        if g + 2 < ngroups:
            if g - 1 >= 0:
                out_pend.pop(g - 1).wait()
            pend[g + 2] = issue(g + 2)
        gc, vc = pend.pop(g)
        gc.wait()
        vc.wait()
        for r in range(_GROUP):
            mvec = bg_v[g * _GROUP + r, :]
            inv = 1.0 - mvec

            def body(c, _):
                sl = pl.ds(c * 16, 16)
                bank_v[b, r, sl] = (vid_v[b, r, sl] * inv
                                    + bank_v[b, r, sl] * mvec)
                return 0

            lax.fori_loop(0, HID // 16, body, 0, unroll=8)
        out_pend[g] = pltpu.async_copy(
            bank_v.at[b], out_hbm.at[pl.ds(base + g * _GROUP, _GROUP)],
            osem[b])
    for g in sorted(out_pend):
        out_pend[g].wait()


@functools.lru_cache(maxsize=1)
def _gather_blend():
    return functools.partial(
        pl.kernel,
        mesh=plsc.VectorSubcoreMesh(core_axis_name="c", subcore_axis_name="s"),
        out_type=jax.ShapeDtypeStruct((NSLOT, HID), jnp.float32),
        scratch_types=[
            pltpu.VMEM((_ROWS_PER_W // _GROUP, _GROUP), jnp.int32),
            pltpu.VMEM((_ROWS_PER_W, 16), jnp.float32),
            pltpu.VMEM((_NBUF, _GROUP, HID), jnp.float32),
            pltpu.VMEM((_NBUF, _GROUP, HID), jnp.float32),
        ] + [pltpu.SemaphoreType.DMA] * 9,
    )(_gather_blend_body)


def kernel(bg_mask, vid_feats, vid_idx, mem_bank):
    bs, v_len, hid = vid_feats.shape
    n_bank = mem_bank.shape[0]
    vid32 = vid_idx.astype(jnp.int32)
    sc_si = _sc_sample()(vid32)
    tc_si = _sample_call(vid32)
    lin = jnp.concatenate([sc_si, tc_si.reshape(_TC_S * bs)])
    sample_flat = lin.reshape(v_len, bs).T.reshape(bs * v_len)
    bank_flat = mem_bank.reshape(n_bank * v_len, hid)
    vid_flat = vid_feats.reshape(bs * v_len, hid)
    bg_wide = jnp.broadcast_to(bg_mask.reshape(bs * v_len, 1), (bs * v_len, 16))
    sample_2d = sample_flat.reshape(bs * v_len // _GROUP, _GROUP)
    out = _gather_blend()(bank_flat, sample_2d, vid_flat, bg_wide)
    return out.reshape(bs, v_len, hid)
